# Initial kernel scaffold; baseline (speedup 1.0000x reference)
import jax
import jax.numpy as jnp
from jax.experimental import pallas as pl


def kernel(x, edge_index, edge_attr, batch, edge_batch, W1, b1, W2, b2, w_vz, b_vz):
    raise NotImplementedError("write your pallas kernel here")


# speedup vs baseline: 4.1956x; 4.1956x over previous
import functools

import jax
import jax.numpy as jnp
from jax import lax
from jax.experimental import pallas as pl
from jax.experimental.pallas import tpu as pltpu
from jax.experimental.pallas import tpu_sc as plsc

N_NODES = 10000
N_EDGES = 160000
D_FEAT = 256
D_EDGE = 16
N_GRAPHS = 64

NC = 2
NS = 16
NW = NC * NS

XB = 80
NX_BLOCKS = N_NODES // XB
EBLK = 640
NE_BLOCKS = N_EDGES // EBLK


def _sc_segment_sums(x, batch, edge_flat, edge_batch, zx, ze):
  mesh = plsc.VectorSubcoreMesh(
      core_axis_name="c", subcore_axis_name="s", num_cores=NC,
      num_subcores=NS)

  @functools.partial(
      pl.kernel,
      out_type=[
          jax.ShapeDtypeStruct((NW, N_GRAPHS, D_FEAT), jnp.float32),
          jax.ShapeDtypeStruct((NW, N_GRAPHS * D_EDGE), jnp.float32),
          jax.ShapeDtypeStruct((NW, N_GRAPHS * 16), jnp.float32),
          jax.ShapeDtypeStruct((NW, N_GRAPHS * 16), jnp.float32),
      ],
      mesh=mesh,
      scratch_types=[
          pltpu.VMEM((XB, D_FEAT), jnp.float32),
          pltpu.VMEM((EBLK * D_EDGE,), jnp.float32),
          pltpu.VMEM((XB,), jnp.int32),
          pltpu.VMEM((EBLK,), jnp.int32),
          pltpu.VMEM((N_GRAPHS, D_FEAT), jnp.float32),
          pltpu.VMEM((N_GRAPHS * D_EDGE,), jnp.float32),
          pltpu.VMEM((N_GRAPHS * 16,), jnp.float32),
          pltpu.VMEM((N_GRAPHS * 16,), jnp.float32),
      ],
  )
  def k(x_hbm, b_hbm, e_hbm, eb_hbm, zx_hbm, ze_hbm,
        sumx_out, sume_out, cntx_out, cnte_out,
        xbuf, ebuf, xids, eids, accx, acce, accx_c, acce_c):
    cid = lax.axis_index("c")
    sid = lax.axis_index("s")
    wid = cid * NS + sid

    pltpu.sync_copy(zx_hbm, accx)
    pltpu.sync_copy(ze_hbm, acce)
    pltpu.sync_copy(ze_hbm, accx_c)
    pltpu.sync_copy(ze_hbm, acce_c)

    ones16 = jnp.ones((16,), jnp.float32)

    def node_block(it, carry):
      blk = it * NW + wid

      @pl.when(blk < NX_BLOCKS)
      def _():
        base = blk * XB
        pltpu.sync_copy(b_hbm.at[pl.ds(base, XB)], xids)
        pltpu.sync_copy(x_hbm.at[pl.ds(base, XB)], xbuf)

        def group(g, c):
          idvec = xids[pl.ds(g * 16, 16)]
          for l in range(16):
            b = idvec[l]
            r = g * 16 + l
            for kk in range(D_FEAT // 16):
              seg = xbuf[pl.ds(r, 1), pl.ds(kk * 16, 16)].reshape((16,))
              plsc.addupdate(accx.at[b, pl.ds(kk * 16, 16)], seg)
            plsc.addupdate(accx_c.at[pl.ds(b * 16, 16)], ones16)
          return c

        lax.fori_loop(0, XB // 16, group, 0)

      return carry

    lax.fori_loop(0, (NX_BLOCKS + NW - 1) // NW, node_block, 0)

    def edge_block(it, carry):
      blk = it * NW + wid

      @pl.when(blk < NE_BLOCKS)
      def _():
        pltpu.sync_copy(eb_hbm.at[pl.ds(blk * EBLK, EBLK)], eids)
        pltpu.sync_copy(e_hbm.at[pl.ds(blk * EBLK * D_EDGE, EBLK * D_EDGE)],
                        ebuf)

        def group(g, c):
          idvec = eids[pl.ds(g * 16, 16)]
          for l in range(16):
            b = idvec[l]
            seg = ebuf[pl.ds((g * 16 + l) * D_EDGE, 16)]
            plsc.addupdate(acce.at[pl.ds(b * 16, 16)], seg)
            plsc.addupdate(acce_c.at[pl.ds(b * 16, 16)], ones16)
          return c

        lax.fori_loop(0, EBLK // 16, group, 0)

      return carry

    lax.fori_loop(0, (NE_BLOCKS + NW - 1) // NW, edge_block, 0)

    pltpu.sync_copy(accx, sumx_out.at[wid])
    pltpu.sync_copy(acce, sume_out.at[wid])
    pltpu.sync_copy(accx_c, cntx_out.at[wid])
    pltpu.sync_copy(acce_c, cnte_out.at[wid])

  return k(x, batch, edge_flat, edge_batch, zx, ze)


def _mlp_head(sumx_ref, sume_ref, cntx_ref, cnte_ref, w1a_ref, w1b_ref,
              b1_ref, w2_ref, b2_ref, wva_ref, wvb_ref, bv_ref, out_ref):
  sum_x = jnp.sum(sumx_ref[...], axis=0)
  sum_e = jnp.sum(sume_ref[...], axis=0)
  cnt_x = jnp.sum(cntx_ref[...], axis=0)[:, 0:1]
  cnt_e = jnp.sum(cnte_ref[...], axis=0)[:, 0:1]

  pre = (jnp.dot(sum_x * 0.1, w1a_ref[...],
                 preferred_element_type=jnp.float32)
         + jnp.dot(sum_e * 0.05, w1b_ref[...],
                   preferred_element_type=jnp.float32)
         + b1_ref[...])
  h = jnp.where(pre >= 0.0, pre, 0.05 * pre)
  ox = jnp.tanh((jnp.sum(h * w2_ref[...], axis=1, keepdims=True)
                 + b2_ref[0, 0]) / 5.0)

  mean_x = sum_x / jnp.maximum(cnt_x, 1.0)
  mean_e = sum_e / jnp.maximum(cnt_e, 1.0)
  ov = jnp.tanh(jnp.sum(mean_x * wva_ref[...], axis=1, keepdims=True)
                + jnp.sum(mean_e * wvb_ref[...], axis=1, keepdims=True)
                + bv_ref[0, 0])
  out_ref[...] = ox + ov


def kernel(x, edge_index, edge_attr, batch, edge_batch, W1, b1, W2, b2,
           w_vz, b_vz):
  del edge_index
  batch = batch.astype(jnp.int32)
  edge_batch = edge_batch.astype(jnp.int32)
  edge_flat = edge_attr.reshape(-1)
  zx = jnp.zeros((N_GRAPHS, D_FEAT), jnp.float32)
  ze = jnp.zeros((N_GRAPHS * 16,), jnp.float32)

  sumx_p, sume_p, cntx_p, cnte_p = _sc_segment_sums(
      x, batch, edge_flat, edge_batch, zx, ze)

  out = pl.pallas_call(
      _mlp_head,
      out_shape=jax.ShapeDtypeStruct((N_GRAPHS, 1), jnp.float32),
  )(sumx_p, sume_p.reshape(NW, N_GRAPHS, D_EDGE),
    cntx_p.reshape(NW, N_GRAPHS, 16), cnte_p.reshape(NW, N_GRAPHS, 16),
    W1[:D_FEAT, :], W1[D_FEAT:, :], b1.reshape(1, -1),
    W2.reshape(1, -1), b2.reshape(1, 1),
    w_vz[:D_FEAT].reshape(1, -1), w_vz[D_FEAT:].reshape(1, -1),
    jnp.asarray(b_vz, jnp.float32).reshape(1, 1))
  return out


# speedup vs baseline: 4.2889x; 1.0222x over previous
import functools

import jax
import jax.numpy as jnp
from jax import lax
from jax.experimental import pallas as pl
from jax.experimental.pallas import tpu as pltpu
from jax.experimental.pallas import tpu_sc as plsc

N_NODES = 10000
N_EDGES = 160000
D_FEAT = 256
D_EDGE = 16
N_GRAPHS = 64

NC = 2
NS = 16
NW = NC * NS

XB = 80
NX_BLOCKS = N_NODES // XB
EBLK = 640
NE_BLOCKS = N_EDGES // EBLK


def _sc_segment_sums(x, batch, edge_attr, edge_batch, zx, ze):
  mesh = plsc.VectorSubcoreMesh(
      core_axis_name="c", subcore_axis_name="s", num_cores=NC,
      num_subcores=NS)

  @functools.partial(
      pl.kernel,
      out_type=[
          jax.ShapeDtypeStruct((NW, N_GRAPHS, D_FEAT), jnp.float32),
          jax.ShapeDtypeStruct((NW, N_GRAPHS * D_EDGE), jnp.float32),
          jax.ShapeDtypeStruct((NW, N_GRAPHS * 16), jnp.float32),
          jax.ShapeDtypeStruct((NW, N_GRAPHS * 16), jnp.float32),
      ],
      mesh=mesh,
      scratch_types=[
          pltpu.VMEM((XB, D_FEAT), jnp.float32),
          pltpu.VMEM((EBLK, D_EDGE), jnp.float32),
          pltpu.VMEM((XB,), jnp.int32),
          pltpu.VMEM((EBLK,), jnp.int32),
          pltpu.VMEM((N_GRAPHS, D_FEAT), jnp.float32),
          pltpu.VMEM((N_GRAPHS * D_EDGE,), jnp.float32),
          pltpu.VMEM((N_GRAPHS * 16,), jnp.float32),
          pltpu.VMEM((N_GRAPHS * 16,), jnp.float32),
      ],
  )
  def k(x_hbm, b_hbm, e_hbm, eb_hbm, zx_hbm, ze_hbm,
        sumx_out, sume_out, cntx_out, cnte_out,
        xbuf, ebuf, xids, eids, accx, acce, accx_c, acce_c):
    cid = lax.axis_index("c")
    sid = lax.axis_index("s")
    wid = cid * NS + sid

    pltpu.sync_copy(zx_hbm, accx)
    pltpu.sync_copy(ze_hbm, acce)
    pltpu.sync_copy(ze_hbm, accx_c)
    pltpu.sync_copy(ze_hbm, acce_c)

    ones16 = jnp.ones((16,), jnp.float32)

    def node_block(it, carry):
      blk = it * NW + wid

      @pl.when(blk < NX_BLOCKS)
      def _():
        base = blk * XB
        pltpu.sync_copy(b_hbm.at[pl.ds(base, XB)], xids)
        pltpu.sync_copy(x_hbm.at[pl.ds(base, XB)], xbuf)

        def group(g, c):
          idvec = xids[pl.ds(g * 16, 16)]
          for l in range(16):
            b = idvec[l]
            r = g * 16 + l
            for kk in range(D_FEAT // 16):
              seg = xbuf[pl.ds(r, 1), pl.ds(kk * 16, 16)].reshape((16,))
              plsc.addupdate(accx.at[b, pl.ds(kk * 16, 16)], seg)
            plsc.addupdate(accx_c.at[pl.ds(b * 16, 16)], ones16)
          return c

        lax.fori_loop(0, XB // 16, group, 0)

      return carry

    lax.fori_loop(0, (NX_BLOCKS + NW - 1) // NW, node_block, 0)

    def edge_block(it, carry):
      blk = it * NW + wid

      @pl.when(blk < NE_BLOCKS)
      def _():
        pltpu.sync_copy(eb_hbm.at[pl.ds(blk * EBLK, EBLK)], eids)
        pltpu.sync_copy(e_hbm.at[pl.ds(blk * EBLK, EBLK)], ebuf)

        def group(g, c):
          idvec = eids[pl.ds(g * 16, 16)]
          for l in range(16):
            b = idvec[l]
            seg = ebuf[pl.ds(g * 16 + l, 1), :].reshape((16,))
            plsc.addupdate(acce.at[pl.ds(b * 16, 16)], seg)
            plsc.addupdate(acce_c.at[pl.ds(b * 16, 16)], ones16)
          return c

        lax.fori_loop(0, EBLK // 16, group, 0)

      return carry

    lax.fori_loop(0, (NE_BLOCKS + NW - 1) // NW, edge_block, 0)

    pltpu.sync_copy(accx, sumx_out.at[wid])
    pltpu.sync_copy(acce, sume_out.at[wid])
    pltpu.sync_copy(accx_c, cntx_out.at[wid])
    pltpu.sync_copy(acce_c, cnte_out.at[wid])

  return k(x, batch, edge_attr, edge_batch, zx, ze)


def _mlp_head(sumx_ref, sume_ref, cntx_ref, cnte_ref, w1a_ref, w1b_ref,
              b1_ref, w2_ref, b2_ref, wva_ref, wvb_ref, bv_ref, out_ref):
  sum_x = jnp.sum(sumx_ref[...], axis=0)
  sum_e = jnp.sum(sume_ref[...], axis=0)
  cnt_x = jnp.sum(cntx_ref[...], axis=0)[:, 0:1]
  cnt_e = jnp.sum(cnte_ref[...], axis=0)[:, 0:1]

  pre = (jnp.dot(sum_x * 0.1, w1a_ref[...],
                 preferred_element_type=jnp.float32)
         + jnp.dot(sum_e * 0.05, w1b_ref[...],
                   preferred_element_type=jnp.float32)
         + b1_ref[...])
  h = jnp.where(pre >= 0.0, pre, 0.05 * pre)
  ox = jnp.tanh((jnp.sum(h * w2_ref[...], axis=1, keepdims=True)
                 + b2_ref[0, 0]) / 5.0)

  mean_x = sum_x / jnp.maximum(cnt_x, 1.0)
  mean_e = sum_e / jnp.maximum(cnt_e, 1.0)
  ov = jnp.tanh(jnp.sum(mean_x * wva_ref[...], axis=1, keepdims=True)
                + jnp.sum(mean_e * wvb_ref[...], axis=1, keepdims=True)
                + bv_ref[0, 0])
  out_ref[...] = ox + ov


def kernel(x, edge_index, edge_attr, batch, edge_batch, W1, b1, W2, b2,
           w_vz, b_vz):
  del edge_index
  batch = batch.astype(jnp.int32)
  edge_batch = edge_batch.astype(jnp.int32)
  zx = jnp.zeros((N_GRAPHS, D_FEAT), jnp.float32)
  ze = jnp.zeros((N_GRAPHS * 16,), jnp.float32)

  sumx_p, sume_p, cntx_p, cnte_p = _sc_segment_sums(
      x, batch, edge_attr, edge_batch, zx, ze)

  out = pl.pallas_call(
      _mlp_head,
      out_shape=jax.ShapeDtypeStruct((N_GRAPHS, 1), jnp.float32),
  )(sumx_p, sume_p.reshape(NW, N_GRAPHS, D_EDGE),
    cntx_p.reshape(NW, N_GRAPHS, 16), cnte_p.reshape(NW, N_GRAPHS, 16),
    W1[:D_FEAT, :], W1[D_FEAT:, :], b1.reshape(1, -1),
    W2.reshape(1, -1), b2.reshape(1, 1),
    w_vz[:D_FEAT].reshape(1, -1), w_vz[D_FEAT:].reshape(1, -1),
    jnp.asarray(b_vz, jnp.float32).reshape(1, 1))
  return out


# speedup vs baseline: 4.8525x; 1.1314x over previous
import functools

import jax
import jax.numpy as jnp
from jax import lax
from jax.experimental import pallas as pl
from jax.experimental.pallas import tpu as pltpu
from jax.experimental.pallas import tpu_sc as plsc

N_NODES = 10000
N_EDGES = 160000
D_FEAT = 256
D_EDGE = 16
N_GRAPHS = 64

NC = 2
NS = 16
NW = NC * NS

XB = 80
NX_BLOCKS = N_NODES // XB
EBLK = 640
NE_BLOCKS = N_EDGES // EBLK


def _sc_segment_sums(x, batch, edge_attr, edge_batch, zx, ze):
  mesh = plsc.VectorSubcoreMesh(
      core_axis_name="c", subcore_axis_name="s", num_cores=NC,
      num_subcores=NS)

  @functools.partial(
      pl.kernel,
      out_type=[
          jax.ShapeDtypeStruct((NW, N_GRAPHS, D_FEAT), jnp.float32),
          jax.ShapeDtypeStruct((NW, N_GRAPHS * D_EDGE), jnp.float32),
          jax.ShapeDtypeStruct((NW, N_GRAPHS * 16), jnp.float32),
          jax.ShapeDtypeStruct((NW, N_GRAPHS * 16), jnp.float32),
      ],
      mesh=mesh,
      scratch_types=[
          pltpu.VMEM((XB, D_FEAT), jnp.float32),
          pltpu.VMEM((EBLK, D_EDGE), jnp.float32),
          pltpu.VMEM((XB,), jnp.int32),
          pltpu.VMEM((EBLK,), jnp.int32),
          pltpu.VMEM((N_GRAPHS, D_FEAT), jnp.float32),
          pltpu.VMEM((N_GRAPHS * D_EDGE,), jnp.float32),
          pltpu.VMEM((N_GRAPHS * 16,), jnp.float32),
          pltpu.VMEM((N_GRAPHS * 16,), jnp.float32),
      ],
  )
  def k(x_hbm, b_hbm, e_hbm, eb_hbm, zx_hbm, ze_hbm,
        sumx_out, sume_out, cntx_out, cnte_out,
        xbuf, ebuf, xids, eids, accx, acce, accx_c, acce_c):
    cid = lax.axis_index("c")
    sid = lax.axis_index("s")
    wid = cid * NS + sid

    pltpu.sync_copy(zx_hbm, accx)
    pltpu.sync_copy(ze_hbm, acce)
    pltpu.sync_copy(ze_hbm, accx_c)
    pltpu.sync_copy(ze_hbm, acce_c)

    ones16 = jnp.ones((16,), jnp.float32)
    sixteens = jnp.full((16,), 16.0, jnp.float32)

    def node_block(it, carry):
      blk = it * NW + wid

      @pl.when(blk < NX_BLOCKS)
      def _():
        base = blk * XB
        pltpu.sync_copy(b_hbm.at[pl.ds(base, XB)], xids)
        pltpu.sync_copy(x_hbm.at[pl.ds(base, XB)], xbuf)

        def group(g, c):
          idvec = xids[pl.ds(g * 16, 16)]
          uniform = idvec[0] == idvec[15]

          @pl.when(uniform)
          def _():
            b = idvec[0]
            for kk in range(D_FEAT // 16):
              segs = [
                  xbuf[pl.ds(g * 16 + l, 1),
                       pl.ds(kk * 16, 16)].reshape((16,))
                  for l in range(16)
              ]
              while len(segs) > 1:
                segs = [a + b2 for a, b2 in zip(segs[::2], segs[1::2])]
              plsc.addupdate(accx.at[b, pl.ds(kk * 16, 16)], segs[0])
            plsc.addupdate(accx_c.at[pl.ds(b * 16, 16)], sixteens)

          @pl.when(jnp.logical_not(uniform))
          def _():
            for l in range(16):
              b = idvec[l]
              r = g * 16 + l
              for kk in range(D_FEAT // 16):
                seg = xbuf[pl.ds(r, 1), pl.ds(kk * 16, 16)].reshape((16,))
                plsc.addupdate(accx.at[b, pl.ds(kk * 16, 16)], seg)
              plsc.addupdate(accx_c.at[pl.ds(b * 16, 16)], ones16)

          return c

        lax.fori_loop(0, XB // 16, group, 0)

      return carry

    lax.fori_loop(0, (NX_BLOCKS + NW - 1) // NW, node_block, 0)

    def edge_block(it, carry):
      blk = it * NW + wid

      @pl.when(blk < NE_BLOCKS)
      def _():
        pltpu.sync_copy(eb_hbm.at[pl.ds(blk * EBLK, EBLK)], eids)
        pltpu.sync_copy(e_hbm.at[pl.ds(blk * EBLK, EBLK)], ebuf)

        def group(g, c):
          idvec = eids[pl.ds(g * 16, 16)]
          uniform = idvec[0] == idvec[15]

          @pl.when(uniform)
          def _():
            b = idvec[0]
            segs = [
                ebuf[pl.ds(g * 16 + l, 1), :].reshape((16,))
                for l in range(16)
            ]
            while len(segs) > 1:
              segs = [a + b2 for a, b2 in zip(segs[::2], segs[1::2])]
            plsc.addupdate(acce.at[pl.ds(b * 16, 16)], segs[0])
            plsc.addupdate(acce_c.at[pl.ds(b * 16, 16)], sixteens)

          @pl.when(jnp.logical_not(uniform))
          def _():
            for l in range(16):
              b = idvec[l]
              seg = ebuf[pl.ds(g * 16 + l, 1), :].reshape((16,))
              plsc.addupdate(acce.at[pl.ds(b * 16, 16)], seg)
              plsc.addupdate(acce_c.at[pl.ds(b * 16, 16)], ones16)

          return c

        lax.fori_loop(0, EBLK // 16, group, 0)

      return carry

    lax.fori_loop(0, (NE_BLOCKS + NW - 1) // NW, edge_block, 0)

    pltpu.sync_copy(accx, sumx_out.at[wid])
    pltpu.sync_copy(acce, sume_out.at[wid])
    pltpu.sync_copy(accx_c, cntx_out.at[wid])
    pltpu.sync_copy(acce_c, cnte_out.at[wid])

  return k(x, batch, edge_attr, edge_batch, zx, ze)


def _mlp_head(sumx_ref, sume_ref, cntx_ref, cnte_ref, w1a_ref, w1b_ref,
              b1_ref, w2_ref, b2_ref, wva_ref, wvb_ref, bv_ref, out_ref):
  sum_x = jnp.sum(sumx_ref[...], axis=0)
  sum_e = jnp.sum(sume_ref[...], axis=0)
  cnt_x = jnp.sum(cntx_ref[...], axis=0)[:, 0:1]
  cnt_e = jnp.sum(cnte_ref[...], axis=0)[:, 0:1]

  pre = (jnp.dot(sum_x * 0.1, w1a_ref[...],
                 preferred_element_type=jnp.float32)
         + jnp.dot(sum_e * 0.05, w1b_ref[...],
                   preferred_element_type=jnp.float32)
         + b1_ref[...])
  h = jnp.where(pre >= 0.0, pre, 0.05 * pre)
  ox = jnp.tanh((jnp.sum(h * w2_ref[...], axis=1, keepdims=True)
                 + b2_ref[0, 0]) / 5.0)

  mean_x = sum_x / jnp.maximum(cnt_x, 1.0)
  mean_e = sum_e / jnp.maximum(cnt_e, 1.0)
  ov = jnp.tanh(jnp.sum(mean_x * wva_ref[...], axis=1, keepdims=True)
                + jnp.sum(mean_e * wvb_ref[...], axis=1, keepdims=True)
                + bv_ref[0, 0])
  out_ref[...] = ox + ov


def kernel(x, edge_index, edge_attr, batch, edge_batch, W1, b1, W2, b2,
           w_vz, b_vz):
  del edge_index
  batch = batch.astype(jnp.int32)
  edge_batch = edge_batch.astype(jnp.int32)
  zx = jnp.zeros((N_GRAPHS, D_FEAT), jnp.float32)
  ze = jnp.zeros((N_GRAPHS * 16,), jnp.float32)

  sumx_p, sume_p, cntx_p, cnte_p = _sc_segment_sums(
      x, batch, edge_attr, edge_batch, zx, ze)

  out = pl.pallas_call(
      _mlp_head,
      out_shape=jax.ShapeDtypeStruct((N_GRAPHS, 1), jnp.float32),
  )(sumx_p, sume_p.reshape(NW, N_GRAPHS, D_EDGE),
    cntx_p.reshape(NW, N_GRAPHS, 16), cnte_p.reshape(NW, N_GRAPHS, 16),
    W1[:D_FEAT, :], W1[D_FEAT:, :], b1.reshape(1, -1),
    W2.reshape(1, -1), b2.reshape(1, 1),
    w_vz[:D_FEAT].reshape(1, -1), w_vz[D_FEAT:].reshape(1, -1),
    jnp.asarray(b_vz, jnp.float32).reshape(1, 1))
  return out


# speedup vs baseline: 4.9633x; 1.0228x over previous
import functools

import jax
import jax.numpy as jnp
from jax import lax
from jax.experimental import pallas as pl
from jax.experimental.pallas import tpu as pltpu
from jax.experimental.pallas import tpu_sc as plsc

N_NODES = 10000
N_EDGES = 160000
D_FEAT = 256
D_EDGE = 16
N_GRAPHS = 64

NC = 2
NS = 16
NW = NC * NS

XB = 80
NX_BLOCKS = N_NODES // XB
NX_FULL = NX_BLOCKS // NW
NX_TAIL = NX_BLOCKS - NX_FULL * NW
EBLK = 256
NE_BLOCKS = N_EDGES // EBLK
NE_FULL = NE_BLOCKS // NW
NE_TAIL = NE_BLOCKS - NE_FULL * NW


def _sc_segment_sums(x, batch, edge_attr, edge_batch, zx, ze):
  mesh = plsc.VectorSubcoreMesh(
      core_axis_name="c", subcore_axis_name="s", num_cores=NC,
      num_subcores=NS)

  @functools.partial(
      pl.kernel,
      out_type=[
          jax.ShapeDtypeStruct((NW, N_GRAPHS, D_FEAT), jnp.float32),
          jax.ShapeDtypeStruct((NW, N_GRAPHS * D_EDGE), jnp.float32),
          jax.ShapeDtypeStruct((NW, N_GRAPHS * 16), jnp.float32),
          jax.ShapeDtypeStruct((NW, N_GRAPHS * 16), jnp.float32),
      ],
      mesh=mesh,
      scratch_types=[
          pltpu.VMEM((XB, D_FEAT), jnp.float32),
          pltpu.VMEM((XB, D_FEAT), jnp.float32),
          pltpu.VMEM((EBLK, D_EDGE), jnp.float32),
          pltpu.VMEM((EBLK, D_EDGE), jnp.float32),
          pltpu.VMEM((XB,), jnp.int32),
          pltpu.VMEM((XB,), jnp.int32),
          pltpu.VMEM((EBLK,), jnp.int32),
          pltpu.VMEM((EBLK,), jnp.int32),
          pltpu.VMEM((N_GRAPHS, D_FEAT), jnp.float32),
          pltpu.VMEM((N_GRAPHS * D_EDGE,), jnp.float32),
          pltpu.VMEM((N_GRAPHS * 16,), jnp.float32),
          pltpu.VMEM((N_GRAPHS * 16,), jnp.float32),
          pltpu.SemaphoreType.DMA,
          pltpu.SemaphoreType.DMA,
          pltpu.SemaphoreType.DMA,
          pltpu.SemaphoreType.DMA,
      ],
  )
  def k(x_hbm, b_hbm, e_hbm, eb_hbm, zx_hbm, ze_hbm,
        sumx_out, sume_out, cntx_out, cnte_out,
        xbuf0, xbuf1, ebuf0, ebuf1, xids0, xids1, eids0, eids1,
        accx, acce, accx_c, acce_c, semi0, semi1, semr0, semr1):
    cid = lax.axis_index("c")
    sid = lax.axis_index("s")
    wid = cid * NS + sid

    xbufs, xidss = (xbuf0, xbuf1), (xids0, xids1)
    ebufs, eidss = (ebuf0, ebuf1), (eids0, eids1)
    semis, semrs = (semi0, semi1), (semr0, semr1)

    dz = [pltpu.async_copy(zx_hbm, accx, semr0),
          pltpu.async_copy(ze_hbm, acce, semr1),
          pltpu.async_copy(ze_hbm, accx_c, semi0),
          pltpu.async_copy(ze_hbm, acce_c, semi1)]

    ones16 = jnp.ones((16,), jnp.float32)
    sixteens = jnp.full((16,), 16.0, jnp.float32)

    def node_compute(xids, xbuf):
      def group(g, c):
        idvec = xids[pl.ds(g * 16, 16)]
        uniform = idvec[0] == idvec[15]

        @pl.when(uniform)
        def _():
          b = idvec[0]
          for kk in range(D_FEAT // 16):
            segs = [
                xbuf[pl.ds(g * 16 + l, 1),
                     pl.ds(kk * 16, 16)].reshape((16,))
                for l in range(16)
            ]
            while len(segs) > 1:
              segs = [a + b2 for a, b2 in zip(segs[::2], segs[1::2])]
            plsc.addupdate(accx.at[b, pl.ds(kk * 16, 16)], segs[0])
          plsc.addupdate(accx_c.at[pl.ds(b * 16, 16)], sixteens)

        @pl.when(jnp.logical_not(uniform))
        def _():
          for l in range(16):
            b = idvec[l]
            for kk in range(D_FEAT // 16):
              seg = xbuf[pl.ds(g * 16 + l, 1),
                         pl.ds(kk * 16, 16)].reshape((16,))
              plsc.addupdate(accx.at[b, pl.ds(kk * 16, 16)], seg)
            plsc.addupdate(accx_c.at[pl.ds(b * 16, 16)], ones16)

        return c

      lax.fori_loop(0, XB // 16, group, 0)

    def edge_compute(eids, ebuf):
      def group(g, c):
        idvec = eids[pl.ds(g * 16, 16)]
        uniform = idvec[0] == idvec[15]

        @pl.when(uniform)
        def _():
          b = idvec[0]
          segs = [
              ebuf[pl.ds(g * 16 + l, 1), :].reshape((16,))
              for l in range(16)
          ]
          while len(segs) > 1:
            segs = [a + b2 for a, b2 in zip(segs[::2], segs[1::2])]
          plsc.addupdate(acce.at[pl.ds(b * 16, 16)], segs[0])
          plsc.addupdate(acce_c.at[pl.ds(b * 16, 16)], sixteens)

        @pl.when(jnp.logical_not(uniform))
        def _():
          for l in range(16):
            b = idvec[l]
            seg = ebuf[pl.ds(g * 16 + l, 1), :].reshape((16,))
            plsc.addupdate(acce.at[pl.ds(b * 16, 16)], seg)
            plsc.addupdate(acce_c.at[pl.ds(b * 16, 16)], ones16)

        return c

      lax.fori_loop(0, EBLK // 16, group, 0)

    def start_node(i):
      buf = i % 2
      base = (i * NW + wid) * XB
      return (pltpu.async_copy(b_hbm.at[pl.ds(base, XB)], xidss[buf],
                               semis[buf]),
              pltpu.async_copy(x_hbm.at[pl.ds(base, XB)], xbufs[buf],
                               semrs[buf]))

    for d in dz:
      d.wait()
    descs = {0: start_node(0)}
    for i in range(NX_FULL):
      if i + 1 < NX_FULL:
        descs[i + 1] = start_node(i + 1)
      d1, d2 = descs.pop(i)
      d1.wait()
      d2.wait()
      node_compute(xidss[i % 2], xbufs[i % 2])

    @pl.when(wid < NX_TAIL)
    def _():
      base = (NX_FULL * NW + wid) * XB
      pltpu.sync_copy(b_hbm.at[pl.ds(base, XB)], xids0)
      pltpu.sync_copy(x_hbm.at[pl.ds(base, XB)], xbuf0)
      node_compute(xids0, xbuf0)

    def start_edge(i):
      buf = i % 2
      base = (i * NW + wid) * EBLK
      return (pltpu.async_copy(eb_hbm.at[pl.ds(base, EBLK)], eidss[buf],
                               semis[buf]),
              pltpu.async_copy(e_hbm.at[pl.ds(base, EBLK)], ebufs[buf],
                               semrs[buf]))

    descs = {0: start_edge(0)}
    for i in range(NE_FULL):
      if i + 1 < NE_FULL:
        descs[i + 1] = start_edge(i + 1)
      d1, d2 = descs.pop(i)
      d1.wait()
      d2.wait()
      edge_compute(eidss[i % 2], ebufs[i % 2])

    @pl.when(wid < NE_TAIL)
    def _():
      base = (NE_FULL * NW + wid) * EBLK
      pltpu.sync_copy(eb_hbm.at[pl.ds(base, EBLK)], eids0)
      pltpu.sync_copy(e_hbm.at[pl.ds(base, EBLK)], ebuf0)
      edge_compute(eids0, ebuf0)

    pltpu.sync_copy(accx, sumx_out.at[wid])
    pltpu.sync_copy(acce, sume_out.at[wid])
    pltpu.sync_copy(accx_c, cntx_out.at[wid])
    pltpu.sync_copy(acce_c, cnte_out.at[wid])

  return k(x, batch, edge_attr, edge_batch, zx, ze)


def _mlp_head(sumx_ref, sume_ref, cntx_ref, cnte_ref, w1a_ref, w1b_ref,
              b1_ref, w2_ref, b2_ref, wva_ref, wvb_ref, bv_ref, out_ref):
  sum_x = jnp.sum(sumx_ref[...], axis=0)
  sum_e = jnp.sum(sume_ref[...], axis=0)
  cnt_x = jnp.sum(cntx_ref[...], axis=0)[:, 0:1]
  cnt_e = jnp.sum(cnte_ref[...], axis=0)[:, 0:1]

  pre = (jnp.dot(sum_x * 0.1, w1a_ref[...],
                 preferred_element_type=jnp.float32)
         + jnp.dot(sum_e * 0.05, w1b_ref[...],
                   preferred_element_type=jnp.float32)
         + b1_ref[...])
  h = jnp.where(pre >= 0.0, pre, 0.05 * pre)
  ox = jnp.tanh((jnp.sum(h * w2_ref[...], axis=1, keepdims=True)
                 + b2_ref[0, 0]) / 5.0)

  mean_x = sum_x / jnp.maximum(cnt_x, 1.0)
  mean_e = sum_e / jnp.maximum(cnt_e, 1.0)
  ov = jnp.tanh(jnp.sum(mean_x * wva_ref[...], axis=1, keepdims=True)
                + jnp.sum(mean_e * wvb_ref[...], axis=1, keepdims=True)
                + bv_ref[0, 0])
  out_ref[...] = ox + ov


def kernel(x, edge_index, edge_attr, batch, edge_batch, W1, b1, W2, b2,
           w_vz, b_vz):
  del edge_index
  batch = batch.astype(jnp.int32)
  edge_batch = edge_batch.astype(jnp.int32)
  zx = jnp.zeros((N_GRAPHS, D_FEAT), jnp.float32)
  ze = jnp.zeros((N_GRAPHS * 16,), jnp.float32)

  sumx_p, sume_p, cntx_p, cnte_p = _sc_segment_sums(
      x, batch, edge_attr, edge_batch, zx, ze)

  out = pl.pallas_call(
      _mlp_head,
      out_shape=jax.ShapeDtypeStruct((N_GRAPHS, 1), jnp.float32),
  )(sumx_p, sume_p.reshape(NW, N_GRAPHS, D_EDGE),
    cntx_p.reshape(NW, N_GRAPHS, 16), cnte_p.reshape(NW, N_GRAPHS, 16),
    W1[:D_FEAT, :], W1[D_FEAT:, :], b1.reshape(1, -1),
    W2.reshape(1, -1), b2.reshape(1, 1),
    w_vz[:D_FEAT].reshape(1, -1), w_vz[D_FEAT:].reshape(1, -1),
    jnp.asarray(b_vz, jnp.float32).reshape(1, 1))
  return out


# speedup vs baseline: 8.5848x; 1.7297x over previous
import functools

import jax
import jax.numpy as jnp
from jax import lax
from jax.experimental import pallas as pl
from jax.experimental.pallas import tpu as pltpu
from jax.experimental.pallas import tpu_sc as plsc

N_NODES = 10000
N_EDGES = 160000
D_FEAT = 256
D_EDGE = 16
N_GRAPHS = 64

NC = 2
NS = 16
NW = NC * NS

XB = 80
NX_BLOCKS = N_NODES // XB
NX_FULL = NX_BLOCKS // NW
NX_TAIL = NX_BLOCKS - NX_FULL * NW
EBLK = 256
NE_BLOCKS = N_EDGES // EBLK
NE_FULL = NE_BLOCKS // NW
NE_TAIL = NE_BLOCKS - NE_FULL * NW


def _sc_segment_sums(x, batch, edge_attr_t, edge_batch, zx, ze, ze3):
  mesh = plsc.VectorSubcoreMesh(
      core_axis_name="c", subcore_axis_name="s", num_cores=NC,
      num_subcores=NS)

  @functools.partial(
      pl.kernel,
      out_type=[
          jax.ShapeDtypeStruct((NW, N_GRAPHS, D_FEAT), jnp.float32),
          jax.ShapeDtypeStruct((NW, N_GRAPHS * D_EDGE * 16), jnp.float32),
          jax.ShapeDtypeStruct((NW, N_GRAPHS * 16), jnp.float32),
          jax.ShapeDtypeStruct((NW, N_GRAPHS * 16), jnp.float32),
      ],
      mesh=mesh,
      scratch_types=[
          pltpu.VMEM((XB, D_FEAT), jnp.float32),
          pltpu.VMEM((XB, D_FEAT), jnp.float32),
          pltpu.VMEM((D_EDGE, EBLK), jnp.float32),
          pltpu.VMEM((D_EDGE, EBLK), jnp.float32),
          pltpu.VMEM((XB,), jnp.int32),
          pltpu.VMEM((XB,), jnp.int32),
          pltpu.VMEM((EBLK,), jnp.int32),
          pltpu.VMEM((EBLK,), jnp.int32),
          pltpu.VMEM((N_GRAPHS, D_FEAT), jnp.float32),
          pltpu.VMEM((N_GRAPHS * D_EDGE * 16,), jnp.float32),
          pltpu.VMEM((N_GRAPHS * 16,), jnp.float32),
          pltpu.VMEM((N_GRAPHS * 16,), jnp.float32),
          pltpu.SemaphoreType.DMA,
          pltpu.SemaphoreType.DMA,
          pltpu.SemaphoreType.DMA,
          pltpu.SemaphoreType.DMA,
          pltpu.SemaphoreType.DMA,
      ],
  )
  def k(x_hbm, b_hbm, e_hbm, eb_hbm, zx_hbm, ze_hbm, ze3_hbm,
        sumx_out, sume3_out, cntx_out, cnte_out,
        xbuf0, xbuf1, ebuf0, ebuf1, xids0, xids1, eids0, eids1,
        accx, acce3, accx_c, acce_c,
        semi0, semi1, semr0, semr1, semz):
    cid = lax.axis_index("c")
    sid = lax.axis_index("s")
    wid = cid * NS + sid

    xbufs, xidss = (xbuf0, xbuf1), (xids0, xids1)
    ebufs, eidss = (ebuf0, ebuf1), (eids0, eids1)
    semis, semrs = (semi0, semi1), (semr0, semr1)

    dz = [pltpu.async_copy(zx_hbm, accx, semr0),
          pltpu.async_copy(ze3_hbm, acce3, semz),
          pltpu.async_copy(ze_hbm, accx_c, semi0),
          pltpu.async_copy(ze_hbm, acce_c, semi1)]

    ones16 = jnp.ones((16,), jnp.float32)
    sixteens = jnp.full((16,), 16.0, jnp.float32)
    iota16 = lax.iota(jnp.int32, 16)
    onehot0 = jnp.where(iota16 == 0, 1.0, 0.0)

    def node_compute(xids, xbuf):
      def group(g, c):
        idvec = xids[pl.ds(g * 16, 16)]
        uniform = idvec[0] == idvec[15]

        @pl.when(uniform)
        def _():
          b = idvec[0]
          for kk in range(D_FEAT // 16):
            segs = [
                xbuf[pl.ds(g * 16 + l, 1),
                     pl.ds(kk * 16, 16)].reshape((16,))
                for l in range(16)
            ]
            while len(segs) > 1:
              segs = [a + b2 for a, b2 in zip(segs[::2], segs[1::2])]
            plsc.addupdate(accx.at[b, pl.ds(kk * 16, 16)], segs[0])
          plsc.addupdate(accx_c.at[pl.ds(b * 16, 16)], ones16)

        @pl.when(jnp.logical_not(uniform))
        def _():
          for l in range(16):
            b = idvec[l]

            def row_chunk(kk, c2):
              seg = xbuf[pl.ds(g * 16 + l, 1),
                         pl.ds(kk * 16, 16)].reshape((16,))
              plsc.addupdate(accx.at[b, pl.ds(kk * 16, 16)], seg)
              return c2

            lax.fori_loop(0, D_FEAT // 16, row_chunk, 0)
            plsc.addupdate(accx_c.at[pl.ds(b * 16, 16)], onehot0)

        return c

      lax.fori_loop(0, XB // 16, group, 0)

    def edge_compute(eids, ebuf):
      first = eids[pl.ds(0, 16)]
      last = eids[pl.ds(EBLK - 16, 16)]
      uniform = first[0] == last[15]

      @pl.when(uniform)
      def _():
        b = first[0]
        for f in range(D_EDGE):
          segs = [
              ebuf[pl.ds(f, 1), pl.ds(m * 16, 16)].reshape((16,))
              for m in range(EBLK // 16)
          ]
          while len(segs) > 1:
            segs = [a + b2 for a, b2 in zip(segs[::2], segs[1::2])]
          plsc.addupdate(acce3.at[pl.ds(b * 256 + f * 16, 16)], segs[0])
        plsc.addupdate(acce_c.at[pl.ds(b * 16, 16)], sixteens)

      @pl.when(jnp.logical_not(uniform))
      def _():
        def group(g, c):
          idvec = eids[pl.ds(g * 16, 16)]
          guniform = idvec[0] == idvec[15]

          @pl.when(guniform)
          def _():
            b = idvec[0]
            for f in range(D_EDGE):
              v = ebuf[pl.ds(f, 1), pl.ds(g * 16, 16)].reshape((16,))
              plsc.addupdate(acce3.at[pl.ds(b * 256 + f * 16, 16)], v)
            plsc.addupdate(acce_c.at[pl.ds(b * 16, 16)], ones16)

          @pl.when(jnp.logical_not(guniform))
          def _():
            for l in range(16):
              b = idvec[l]
              mask = iota16 == l

              def feat(f, c2):
                v = ebuf[pl.ds(f, 1), pl.ds(g * 16, 16)].reshape((16,))
                plsc.addupdate(acce3.at[pl.ds(b * 256 + f * 16, 16)],
                               jnp.where(mask, v, 0.0))
                return c2

              lax.fori_loop(0, D_EDGE, feat, 0)
              plsc.addupdate(acce_c.at[pl.ds(b * 16, 16)],
                             jnp.where(mask, 1.0, 0.0))

          return c

        lax.fori_loop(0, EBLK // 16, group, 0)

    def start_node(i):
      buf = i % 2
      base = (i * NW + wid) * XB
      return (pltpu.async_copy(b_hbm.at[pl.ds(base, XB)], xidss[buf],
                               semis[buf]),
              pltpu.async_copy(x_hbm.at[pl.ds(base, XB)], xbufs[buf],
                               semrs[buf]))

    for d in dz:
      d.wait()
    descs = {0: start_node(0)}
    for i in range(NX_FULL):
      if i + 1 < NX_FULL:
        descs[i + 1] = start_node(i + 1)
      d1, d2 = descs.pop(i)
      d1.wait()
      d2.wait()
      node_compute(xidss[i % 2], xbufs[i % 2])

    @pl.when(wid < NX_TAIL)
    def _():
      base = (NX_FULL * NW + wid) * XB
      pltpu.sync_copy(b_hbm.at[pl.ds(base, XB)], xids0)
      pltpu.sync_copy(x_hbm.at[pl.ds(base, XB)], xbuf0)
      node_compute(xids0, xbuf0)

    def start_edge(i, buf):
      base = (i * NW + wid) * EBLK
      return (pltpu.async_copy(eb_hbm.at[pl.ds(base, EBLK)], eidss[buf],
                               semis[buf]),
              pltpu.async_copy(e_hbm.at[:, pl.ds(base, EBLK)], ebufs[buf],
                               semrs[buf]))

    def wait_edge(i, buf):
      base = (i * NW + wid) * EBLK
      pltpu.make_async_copy(eb_hbm.at[pl.ds(base, EBLK)], eidss[buf],
                            semis[buf]).wait()
      pltpu.make_async_copy(e_hbm.at[:, pl.ds(base, EBLK)], ebufs[buf],
                            semrs[buf]).wait()

    start_edge(0, 0)
    start_edge(1, 1)

    def pair(p, c):
      i0 = 2 * p
      wait_edge(i0, 0)
      edge_compute(eids0, ebuf0)
      start_edge(i0 + 2, 0)
      i1 = i0 + 1
      wait_edge(i1, 1)
      edge_compute(eids1, ebuf1)

      @pl.when(i1 + 2 < NE_FULL)
      def _():
        start_edge(i1 + 2, 1)

      return c

    lax.fori_loop(0, (NE_FULL - 1) // 2, pair, 0)
    wait_edge(NE_FULL - 1, (NE_FULL - 1) % 2)
    edge_compute(eidss[(NE_FULL - 1) % 2], ebufs[(NE_FULL - 1) % 2])

    @pl.when(wid < NE_TAIL)
    def _():
      base = (NE_FULL * NW + wid) * EBLK
      pltpu.sync_copy(eb_hbm.at[pl.ds(base, EBLK)], eids1)
      pltpu.sync_copy(e_hbm.at[:, pl.ds(base, EBLK)], ebuf1)
      edge_compute(eids1, ebuf1)

    pltpu.sync_copy(accx, sumx_out.at[wid])
    pltpu.sync_copy(acce3, sume3_out.at[wid])
    pltpu.sync_copy(accx_c, cntx_out.at[wid])
    pltpu.sync_copy(acce_c, cnte_out.at[wid])

  return k(x, batch, edge_attr_t, edge_batch, zx, ze, ze3)


def _mlp_head(sumx_ref, sume3_ref, cntx_ref, cnte_ref, w1a_ref, w1br_ref,
              b1_ref, w2_ref, b2_ref, wva_ref, wvbr_ref, bv_ref, out_ref):
  sum_x = jnp.sum(sumx_ref[...], axis=0)
  e3 = jnp.sum(sume3_ref[...], axis=0)
  cnt_x = jnp.sum(jnp.sum(cntx_ref[...], axis=0), axis=1,
                  keepdims=True)
  cnt_e = jnp.sum(jnp.sum(cnte_ref[...], axis=0), axis=1,
                  keepdims=True)

  pre = (jnp.dot(sum_x * 0.1, w1a_ref[...],
                 preferred_element_type=jnp.float32)
         + jnp.dot(e3 * 0.05, w1br_ref[...],
                   preferred_element_type=jnp.float32)
         + b1_ref[...])
  h = jnp.where(pre >= 0.0, pre, 0.05 * pre)
  ox = jnp.tanh((jnp.sum(h * w2_ref[...], axis=1, keepdims=True)
                 + b2_ref[0, 0]) / 5.0)

  mean_x = sum_x / jnp.maximum(cnt_x, 1.0)
  ov = jnp.tanh(jnp.sum(mean_x * wva_ref[...], axis=1, keepdims=True)
                + (jnp.sum(e3 * wvbr_ref[...], axis=1, keepdims=True)
                   / jnp.maximum(cnt_e, 1.0))
                + bv_ref[0, 0])
  out_ref[...] = ox + ov


def kernel(x, edge_index, edge_attr, batch, edge_batch, W1, b1, W2, b2,
           w_vz, b_vz):
  del edge_index
  batch = batch.astype(jnp.int32)
  edge_batch = edge_batch.astype(jnp.int32)
  zx = jnp.zeros((N_GRAPHS, D_FEAT), jnp.float32)
  ze = jnp.zeros((N_GRAPHS * 16,), jnp.float32)
  ze3 = jnp.zeros((N_GRAPHS * D_EDGE * 16,), jnp.float32)

  sumx_p, sume3_p, cntx_p, cnte_p = _sc_segment_sums(
      x, batch, edge_attr.T, edge_batch, zx, ze, ze3)

  w1b_rep = jnp.repeat(W1[D_FEAT:, :], 16, axis=0)
  wvb_rep = jnp.repeat(w_vz[D_FEAT:], 16).reshape(1, -1)

  out = pl.pallas_call(
      _mlp_head,
      out_shape=jax.ShapeDtypeStruct((N_GRAPHS, 1), jnp.float32),
  )(sumx_p, sume3_p.reshape(NW, N_GRAPHS, D_FEAT),
    cntx_p.reshape(NW, N_GRAPHS, 16), cnte_p.reshape(NW, N_GRAPHS, 16),
    W1[:D_FEAT, :], w1b_rep, b1.reshape(1, -1),
    W2.reshape(1, -1), b2.reshape(1, 1),
    w_vz[:D_FEAT].reshape(1, -1), wvb_rep,
    jnp.asarray(b_vz, jnp.float32).reshape(1, 1))
  return out


# speedup vs baseline: 8.6662x; 1.0095x over previous
import functools

import jax
import jax.numpy as jnp
from jax import lax
from jax.experimental import pallas as pl
from jax.experimental.pallas import tpu as pltpu
from jax.experimental.pallas import tpu_sc as plsc

N_NODES = 10000
N_EDGES = 160000
D_FEAT = 256
D_EDGE = 16
N_GRAPHS = 64

NC = 2
NS = 16
NW = NC * NS

XB = 80
NX_BLOCKS = N_NODES // XB
NX_FULL = NX_BLOCKS // NW
NX_TAIL = NX_BLOCKS - NX_FULL * NW
EBLK = 640
NE_BLOCKS = N_EDGES // EBLK
NE_FULL = NE_BLOCKS // NW
NE_TAIL = NE_BLOCKS - NE_FULL * NW


def _sc_segment_sums(x, batch, edge_attr_t, edge_batch, zx, ze):
  mesh = plsc.VectorSubcoreMesh(
      core_axis_name="c", subcore_axis_name="s", num_cores=NC,
      num_subcores=NS)

  @functools.partial(
      pl.kernel,
      out_type=[
          jax.ShapeDtypeStruct((NW, N_GRAPHS, D_FEAT), jnp.float32),
          jax.ShapeDtypeStruct((NW, N_GRAPHS, D_FEAT), jnp.float32),
          jax.ShapeDtypeStruct((NW, N_GRAPHS, 16), jnp.float32),
          jax.ShapeDtypeStruct((NW, N_GRAPHS, 16), jnp.float32),
      ],
      mesh=mesh,
      scratch_types=[
          pltpu.VMEM((XB, D_FEAT), jnp.float32),
          pltpu.VMEM((XB, D_FEAT), jnp.float32),
          pltpu.VMEM((D_EDGE, EBLK), jnp.float32),
          pltpu.VMEM((D_EDGE, EBLK), jnp.float32),
          pltpu.VMEM((XB,), jnp.int32),
          pltpu.VMEM((XB,), jnp.int32),
          pltpu.VMEM((EBLK,), jnp.int32),
          pltpu.VMEM((EBLK,), jnp.int32),
          pltpu.VMEM((N_GRAPHS, D_FEAT), jnp.float32),
          pltpu.VMEM((N_GRAPHS, D_FEAT), jnp.float32),
          pltpu.VMEM((N_GRAPHS, 16), jnp.float32),
          pltpu.VMEM((N_GRAPHS, 16), jnp.float32),
          pltpu.SemaphoreType.DMA,
          pltpu.SemaphoreType.DMA,
          pltpu.SemaphoreType.DMA,
          pltpu.SemaphoreType.DMA,
          pltpu.SemaphoreType.DMA,
      ],
  )
  def k(x_hbm, b_hbm, e_hbm, eb_hbm, zx_hbm, ze_hbm,
        sumx_out, sume3_out, cntx_out, cnte_out,
        xbuf0, xbuf1, ebuf0, ebuf1, xids0, xids1, eids0, eids1,
        accx, acce3, accx_c, acce_c,
        semi0, semi1, semr0, semr1, semz):
    cid = lax.axis_index("c")
    sid = lax.axis_index("s")
    wid = cid * NS + sid

    xbufs, xidss = (xbuf0, xbuf1), (xids0, xids1)
    ebufs, eidss = (ebuf0, ebuf1), (eids0, eids1)
    semis, semrs = (semi0, semi1), (semr0, semr1)

    dz = [pltpu.async_copy(zx_hbm, accx, semr0),
          pltpu.async_copy(zx_hbm, acce3, semz),
          pltpu.async_copy(ze_hbm, accx_c, semi0),
          pltpu.async_copy(ze_hbm, acce_c, semi1)]

    ones16 = jnp.ones((16,), jnp.float32)
    iota16 = lax.iota(jnp.int32, 16)
    onehot0 = jnp.where(iota16 == 0, 1.0, 0.0)

    def node_compute(xids, xbuf):
      def group(g, c):
        idvec = xids[pl.ds(g * 16, 16)]
        uniform = idvec[0] == idvec[15]

        @pl.when(uniform)
        def _():
          b = idvec[0]
          for kk in range(D_FEAT // 16):
            segs = []
            for l in range(8):
              a1 = xbuf[pl.ds(g * 16 + 2 * l, 1),
                        pl.ds(kk * 16, 16)].reshape((16,))
              a2 = xbuf[pl.ds(g * 16 + 2 * l + 1, 1),
                        pl.ds(kk * 16, 16)].reshape((16,))
              segs.append(a1 + a2)
            while len(segs) > 1:
              segs = [a + b2 for a, b2 in zip(segs[::2], segs[1::2])]
            plsc.addupdate(accx.at[b, pl.ds(kk * 16, 16)], segs[0])
          plsc.addupdate(accx_c.at[b], ones16)

        @pl.when(jnp.logical_not(uniform))
        def _():
          for l in range(16):
            b = idvec[l]

            def row_chunk(kk, c2):
              seg = xbuf[pl.ds(g * 16 + l, 1),
                         pl.ds(kk * 16, 16)].reshape((16,))
              plsc.addupdate(accx.at[b, pl.ds(kk * 16, 16)], seg)
              return c2

            lax.fori_loop(0, D_FEAT // 16, row_chunk, 0)
            plsc.addupdate(accx_c.at[b], onehot0)

        return c

      lax.fori_loop(0, XB // 16, group, 0)

    def edge_compute(eids, ebuf):
      first = eids[pl.ds(0, 16)]
      last = eids[pl.ds(EBLK - 16, 16)]
      uniform = first[0] == last[15]

      @pl.when(uniform)
      def _():
        b = first[0]

        def feat(f, c):
          segs = []
          for m in range(EBLK // 32):
            a1 = ebuf[pl.ds(f, 1), pl.ds(2 * m * 16, 16)].reshape((16,))
            a2 = ebuf[pl.ds(f, 1),
                      pl.ds((2 * m + 1) * 16, 16)].reshape((16,))
            segs.append(a1 + a2)
          while len(segs) > 1:
            segs = [a + b2 for a, b2 in zip(segs[::2], segs[1::2])]
          plsc.addupdate(acce3.at[b, pl.ds(f * 16, 16)], segs[0])
          return c

        lax.fori_loop(0, D_EDGE, feat, 0)
        plsc.addupdate(acce_c.at[b], jnp.full((16,), EBLK / 16.0,
                                              jnp.float32))

      @pl.when(jnp.logical_not(uniform))
      def _():
        def group(g, c):
          idvec = eids[pl.ds(g * 16, 16)]
          guniform = idvec[0] == idvec[15]

          @pl.when(guniform)
          def _():
            b = idvec[0]
            for f in range(D_EDGE):
              v = ebuf[pl.ds(f, 1), pl.ds(g * 16, 16)].reshape((16,))
              plsc.addupdate(acce3.at[b, pl.ds(f * 16, 16)], v)
            plsc.addupdate(acce_c.at[b], ones16)

          @pl.when(jnp.logical_not(guniform))
          def _():
            for l in range(16):
              b = idvec[l]
              mask = iota16 == l

              def feat2(f, c2):
                v = ebuf[pl.ds(f, 1), pl.ds(g * 16, 16)].reshape((16,))
                plsc.addupdate(acce3.at[b, pl.ds(f * 16, 16)],
                               jnp.where(mask, v, 0.0))
                return c2

              lax.fori_loop(0, D_EDGE, feat2, 0)
              plsc.addupdate(acce_c.at[b], jnp.where(mask, 1.0, 0.0))

          return c

        lax.fori_loop(0, EBLK // 16, group, 0)

    def start_node(i):
      buf = i % 2
      base = (i * NW + wid) * XB
      return (pltpu.async_copy(b_hbm.at[pl.ds(base, XB)], xidss[buf],
                               semis[buf]),
              pltpu.async_copy(x_hbm.at[pl.ds(base, XB)], xbufs[buf],
                               semrs[buf]))

    for d in dz:
      d.wait()
    descs = {0: start_node(0)}
    for i in range(NX_FULL):
      if i + 1 < NX_FULL:
        descs[i + 1] = start_node(i + 1)
      d1, d2 = descs.pop(i)
      d1.wait()
      d2.wait()
      node_compute(xidss[i % 2], xbufs[i % 2])

    @pl.when(wid < NX_TAIL)
    def _():
      base = (NX_FULL * NW + wid) * XB
      pltpu.sync_copy(b_hbm.at[pl.ds(base, XB)], xids0)
      pltpu.sync_copy(x_hbm.at[pl.ds(base, XB)], xbuf0)
      node_compute(xids0, xbuf0)

    def start_edge(i, buf):
      base = (i * NW + wid) * EBLK
      return (pltpu.async_copy(eb_hbm.at[pl.ds(base, EBLK)], eidss[buf],
                               semis[buf]),
              pltpu.async_copy(e_hbm.at[:, pl.ds(base, EBLK)], ebufs[buf],
                               semrs[buf]))

    def wait_edge(i, buf):
      base = (i * NW + wid) * EBLK
      pltpu.make_async_copy(eb_hbm.at[pl.ds(base, EBLK)], eidss[buf],
                            semis[buf]).wait()
      pltpu.make_async_copy(e_hbm.at[:, pl.ds(base, EBLK)], ebufs[buf],
                            semrs[buf]).wait()

    start_edge(0, 0)
    start_edge(1, 1)

    def pair(p, c):
      i0 = 2 * p
      wait_edge(i0, 0)
      edge_compute(eids0, ebuf0)
      start_edge(i0 + 2, 0)
      i1 = i0 + 1
      wait_edge(i1, 1)
      edge_compute(eids1, ebuf1)

      @pl.when(i1 + 2 < NE_FULL)
      def _():
        start_edge(i1 + 2, 1)

      return c

    lax.fori_loop(0, (NE_FULL - 1) // 2, pair, 0)
    wait_edge(NE_FULL - 1, (NE_FULL - 1) % 2)
    edge_compute(eidss[(NE_FULL - 1) % 2], ebufs[(NE_FULL - 1) % 2])

    @pl.when(wid < NE_TAIL)
    def _():
      base = (NE_FULL * NW + wid) * EBLK
      pltpu.sync_copy(eb_hbm.at[pl.ds(base, EBLK)], eids1)
      pltpu.sync_copy(e_hbm.at[:, pl.ds(base, EBLK)], ebuf1)
      edge_compute(eids1, ebuf1)

    pltpu.sync_copy(accx, sumx_out.at[wid])
    pltpu.sync_copy(acce3, sume3_out.at[wid])
    pltpu.sync_copy(accx_c, cntx_out.at[wid])
    pltpu.sync_copy(acce_c, cnte_out.at[wid])

  return k(x, batch, edge_attr_t, edge_batch, zx, ze)


def _mlp_head(sumx_ref, sume3_ref, cntx_ref, cnte_ref, w1a_ref, w1br_ref,
              b1_ref, w2_ref, b2_ref, wva_ref, wvbr_ref, bv_ref, out_ref):
  sum_x = jnp.sum(sumx_ref[...], axis=0)
  e3 = jnp.sum(sume3_ref[...], axis=0)
  cnt_x = jnp.sum(jnp.sum(cntx_ref[...], axis=0), axis=1,
                  keepdims=True)
  cnt_e = jnp.sum(jnp.sum(cnte_ref[...], axis=0), axis=1,
                  keepdims=True)

  pre = (jnp.dot(sum_x * 0.1, w1a_ref[...],
                 preferred_element_type=jnp.float32)
         + jnp.dot(e3 * 0.05, w1br_ref[...],
                   preferred_element_type=jnp.float32)
         + b1_ref[...])
  h = jnp.where(pre >= 0.0, pre, 0.05 * pre)
  ox = jnp.tanh((jnp.sum(h * w2_ref[...], axis=1, keepdims=True)
                 + b2_ref[0, 0]) / 5.0)

  mean_x = sum_x / jnp.maximum(cnt_x, 1.0)
  ov = jnp.tanh(jnp.sum(mean_x * wva_ref[...], axis=1, keepdims=True)
                + (jnp.sum(e3 * wvbr_ref[...], axis=1, keepdims=True)
                   / jnp.maximum(cnt_e, 1.0))
                + bv_ref[0, 0])
  out_ref[...] = ox + ov


def kernel(x, edge_index, edge_attr, batch, edge_batch, W1, b1, W2, b2,
           w_vz, b_vz):
  del edge_index
  batch = batch.astype(jnp.int32)
  edge_batch = edge_batch.astype(jnp.int32)
  zx = jnp.zeros((N_GRAPHS, D_FEAT), jnp.float32)
  ze = jnp.zeros((N_GRAPHS, 16), jnp.float32)

  sumx_p, sume3_p, cntx_p, cnte_p = _sc_segment_sums(
      x, batch, edge_attr.T, edge_batch, zx, ze)

  w1b_rep = jnp.repeat(W1[D_FEAT:, :], 16, axis=0)
  wvb_rep = jnp.repeat(w_vz[D_FEAT:], 16).reshape(1, -1)

  out = pl.pallas_call(
      _mlp_head,
      out_shape=jax.ShapeDtypeStruct((N_GRAPHS, 1), jnp.float32),
  )(sumx_p, sume3_p, cntx_p, cnte_p,
    W1[:D_FEAT, :], w1b_rep, b1.reshape(1, -1),
    W2.reshape(1, -1), b2.reshape(1, 1),
    w_vz[:D_FEAT].reshape(1, -1), wvb_rep,
    jnp.asarray(b_vz, jnp.float32).reshape(1, 1))
  return out


# speedup vs baseline: 9.0865x; 1.0485x over previous
import functools

import jax
import jax.numpy as jnp
from jax import lax
from jax.experimental import pallas as pl
from jax.experimental.pallas import tpu as pltpu
from jax.experimental.pallas import tpu_sc as plsc

N_NODES = 10000
N_EDGES = 160000
D_FEAT = 256
D_EDGE = 16
N_GRAPHS = 64

NC = 2
NS = 16
NW = NC * NS

XB = 80
NX_BLOCKS = N_NODES // XB
NX_FULL = NX_BLOCKS // NW
NX_TAIL = NX_BLOCKS - NX_FULL * NW
EBLK = 256
NE_BLOCKS = N_EDGES // EBLK
NE_FULL = NE_BLOCKS // NW
NE_TAIL = NE_BLOCKS - NE_FULL * NW


def _sc_segment_sums(x, batch, edge_attr_t, edge_batch, zx, ze):
  mesh = plsc.VectorSubcoreMesh(
      core_axis_name="c", subcore_axis_name="s", num_cores=NC,
      num_subcores=NS)

  @functools.partial(
      pl.kernel,
      out_type=[
          jax.ShapeDtypeStruct((NW, N_GRAPHS, D_FEAT), jnp.float32),
          jax.ShapeDtypeStruct((NW, N_GRAPHS, D_FEAT), jnp.float32),
          jax.ShapeDtypeStruct((NW, N_GRAPHS, 16), jnp.float32),
          jax.ShapeDtypeStruct((NW, N_GRAPHS, 16), jnp.float32),
      ],
      mesh=mesh,
      scratch_types=[
          pltpu.VMEM((XB, D_FEAT), jnp.float32),
          pltpu.VMEM((XB, D_FEAT), jnp.float32),
          pltpu.VMEM((D_EDGE, EBLK), jnp.float32),
          pltpu.VMEM((D_EDGE, EBLK), jnp.float32),
          pltpu.VMEM((XB,), jnp.int32),
          pltpu.VMEM((XB,), jnp.int32),
          pltpu.VMEM((EBLK,), jnp.int32),
          pltpu.VMEM((EBLK,), jnp.int32),
          pltpu.VMEM((N_GRAPHS, D_FEAT), jnp.float32),
          pltpu.VMEM((N_GRAPHS, D_FEAT), jnp.float32),
          pltpu.VMEM((N_GRAPHS, 16), jnp.float32),
          pltpu.VMEM((N_GRAPHS, 16), jnp.float32),
          pltpu.SemaphoreType.DMA,
          pltpu.SemaphoreType.DMA,
          pltpu.SemaphoreType.DMA,
          pltpu.SemaphoreType.DMA,
          pltpu.SemaphoreType.DMA,
      ],
  )
  def k(x_hbm, b_hbm, e_hbm, eb_hbm, zx_hbm, ze_hbm,
        sumx_out, sume3_out, cntx_out, cnte_out,
        xbuf0, xbuf1, ebuf0, ebuf1, xids0, xids1, eids0, eids1,
        accx, acce3, accx_c, acce_c,
        semi0, semi1, semr0, semr1, semz):
    cid = lax.axis_index("c")
    sid = lax.axis_index("s")
    wid = cid * NS + sid

    xbufs, xidss = (xbuf0, xbuf1), (xids0, xids1)
    ebufs, eidss = (ebuf0, ebuf1), (eids0, eids1)
    semis, semrs = (semi0, semi1), (semr0, semr1)

    dz = [pltpu.async_copy(zx_hbm, accx, semr0),
          pltpu.async_copy(zx_hbm, acce3, semz),
          pltpu.async_copy(ze_hbm, accx_c, semi0),
          pltpu.async_copy(ze_hbm, acce_c, semi1)]

    ones16 = jnp.ones((16,), jnp.float32)
    iota16 = lax.iota(jnp.int32, 16)
    onehot0 = jnp.where(iota16 == 0, 1.0, 0.0)

    def node_compute(xids, xbuf):
      def group(g, c):
        idvec = xids[pl.ds(g * 16, 16)]
        uniform = idvec[0] == idvec[15]

        @pl.when(uniform)
        def _():
          b = idvec[0]
          for kk in range(D_FEAT // 16):
            segs = []
            for l in range(8):
              a1 = xbuf[pl.ds(g * 16 + 2 * l, 1),
                        pl.ds(kk * 16, 16)].reshape((16,))
              a2 = xbuf[pl.ds(g * 16 + 2 * l + 1, 1),
                        pl.ds(kk * 16, 16)].reshape((16,))
              segs.append(a1 + a2)
            while len(segs) > 1:
              nxt = [a + b2 for a, b2 in zip(segs[::2], segs[1::2])]
              if len(segs) % 2:
                nxt.append(segs[-1])
              segs = nxt
            plsc.addupdate(accx.at[b, pl.ds(kk * 16, 16)], segs[0])
          plsc.addupdate(accx_c.at[b], ones16)

        @pl.when(jnp.logical_not(uniform))
        def _():
          for l in range(16):
            b = idvec[l]

            def row_chunk(kk, c2):
              seg = xbuf[pl.ds(g * 16 + l, 1),
                         pl.ds(kk * 16, 16)].reshape((16,))
              plsc.addupdate(accx.at[b, pl.ds(kk * 16, 16)], seg)
              return c2

            lax.fori_loop(0, D_FEAT // 16, row_chunk, 0)
            plsc.addupdate(accx_c.at[b], onehot0)

        return c

      lax.fori_loop(0, XB // 16, group, 0)

    def edge_compute(eids, ebuf):
      first = eids[pl.ds(0, 16)]
      last = eids[pl.ds(EBLK - 16, 16)]
      uniform = first[0] == last[15]

      @pl.when(uniform)
      def _():
        b = first[0]

        def feat(f, c):
          segs = []
          for m in range(EBLK // 32):
            a1 = ebuf[pl.ds(f, 1), pl.ds(2 * m * 16, 16)].reshape((16,))
            a2 = ebuf[pl.ds(f, 1),
                      pl.ds((2 * m + 1) * 16, 16)].reshape((16,))
            segs.append(a1 + a2)
          while len(segs) > 1:
            nxt = [a + b2 for a, b2 in zip(segs[::2], segs[1::2])]
            if len(segs) % 2:
              nxt.append(segs[-1])
            segs = nxt
          plsc.addupdate(acce3.at[b, pl.ds(f * 16, 16)], segs[0])
          return c

        lax.fori_loop(0, D_EDGE, feat, 0)
        plsc.addupdate(acce_c.at[b], jnp.full((16,), EBLK / 16.0,
                                              jnp.float32))

      @pl.when(jnp.logical_not(uniform))
      def _():
        def group(g, c):
          idvec = eids[pl.ds(g * 16, 16)]
          guniform = idvec[0] == idvec[15]

          @pl.when(guniform)
          def _():
            b = idvec[0]
            for f in range(D_EDGE):
              v = ebuf[pl.ds(f, 1), pl.ds(g * 16, 16)].reshape((16,))
              plsc.addupdate(acce3.at[b, pl.ds(f * 16, 16)], v)
            plsc.addupdate(acce_c.at[b], ones16)

          @pl.when(jnp.logical_not(guniform))
          def _():
            for l in range(16):
              b = idvec[l]
              mask = iota16 == l

              def feat2(f, c2):
                v = ebuf[pl.ds(f, 1), pl.ds(g * 16, 16)].reshape((16,))
                plsc.addupdate(acce3.at[b, pl.ds(f * 16, 16)],
                               jnp.where(mask, v, 0.0))
                return c2

              lax.fori_loop(0, D_EDGE, feat2, 0)
              plsc.addupdate(acce_c.at[b], jnp.where(mask, 1.0, 0.0))

          return c

        lax.fori_loop(0, EBLK // 16, group, 0)

    def start_node(i):
      buf = i % 2
      base = (i * NW + wid) * XB
      return (pltpu.async_copy(b_hbm.at[pl.ds(base, XB)], xidss[buf],
                               semis[buf]),
              pltpu.async_copy(x_hbm.at[pl.ds(base, XB)], xbufs[buf],
                               semrs[buf]))

    for d in dz:
      d.wait()
    descs = {0: start_node(0)}
    for i in range(NX_FULL):
      if i + 1 < NX_FULL:
        descs[i + 1] = start_node(i + 1)
      d1, d2 = descs.pop(i)
      d1.wait()
      d2.wait()
      node_compute(xidss[i % 2], xbufs[i % 2])

    @pl.when(wid < NX_TAIL)
    def _():
      base = (NX_FULL * NW + wid) * XB
      pltpu.sync_copy(b_hbm.at[pl.ds(base, XB)], xids0)
      pltpu.sync_copy(x_hbm.at[pl.ds(base, XB)], xbuf0)
      node_compute(xids0, xbuf0)

    def start_edge(i, buf):
      base = (i * NW + wid) * EBLK
      return (pltpu.async_copy(eb_hbm.at[pl.ds(base, EBLK)], eidss[buf],
                               semis[buf]),
              pltpu.async_copy(e_hbm.at[:, pl.ds(base, EBLK)], ebufs[buf],
                               semrs[buf]))

    def wait_edge(i, buf):
      base = (i * NW + wid) * EBLK
      pltpu.make_async_copy(eb_hbm.at[pl.ds(base, EBLK)], eidss[buf],
                            semis[buf]).wait()
      pltpu.make_async_copy(e_hbm.at[:, pl.ds(base, EBLK)], ebufs[buf],
                            semrs[buf]).wait()

    start_edge(0, 0)
    start_edge(1, 1)

    def pair(p, c):
      i0 = 2 * p
      wait_edge(i0, 0)
      edge_compute(eids0, ebuf0)
      start_edge(i0 + 2, 0)
      i1 = i0 + 1
      wait_edge(i1, 1)
      edge_compute(eids1, ebuf1)

      @pl.when(i1 + 2 < NE_FULL)
      def _():
        start_edge(i1 + 2, 1)

      return c

    lax.fori_loop(0, (NE_FULL - 1) // 2, pair, 0)
    wait_edge(NE_FULL - 1, (NE_FULL - 1) % 2)
    edge_compute(eidss[(NE_FULL - 1) % 2], ebufs[(NE_FULL - 1) % 2])

    @pl.when(wid < NE_TAIL)
    def _():
      base = (NE_FULL * NW + wid) * EBLK
      pltpu.sync_copy(eb_hbm.at[pl.ds(base, EBLK)], eids1)
      pltpu.sync_copy(e_hbm.at[:, pl.ds(base, EBLK)], ebuf1)
      edge_compute(eids1, ebuf1)

    pltpu.sync_copy(accx, sumx_out.at[wid])
    pltpu.sync_copy(acce3, sume3_out.at[wid])
    pltpu.sync_copy(accx_c, cntx_out.at[wid])
    pltpu.sync_copy(acce_c, cnte_out.at[wid])

  return k(x, batch, edge_attr_t, edge_batch, zx, ze)


def _mlp_head(sumx_ref, sume3_ref, cntx_ref, cnte_ref, w1a_ref, w1br_ref,
              b1_ref, w2_ref, b2_ref, wva_ref, wvbr_ref, bv_ref, out_ref):
  sum_x = jnp.sum(sumx_ref[...], axis=0)
  e3 = jnp.sum(sume3_ref[...], axis=0)
  cnt_x = jnp.sum(jnp.sum(cntx_ref[...], axis=0), axis=1,
                  keepdims=True)
  cnt_e = jnp.sum(jnp.sum(cnte_ref[...], axis=0), axis=1,
                  keepdims=True)

  pre = (jnp.dot(sum_x * 0.1, w1a_ref[...],
                 preferred_element_type=jnp.float32)
         + jnp.dot(e3 * 0.05, w1br_ref[...],
                   preferred_element_type=jnp.float32)
         + b1_ref[...])
  h = jnp.where(pre >= 0.0, pre, 0.05 * pre)
  ox = jnp.tanh((jnp.sum(h * w2_ref[...], axis=1, keepdims=True)
                 + b2_ref[0, 0]) / 5.0)

  mean_x = sum_x / jnp.maximum(cnt_x, 1.0)
  ov = jnp.tanh(jnp.sum(mean_x * wva_ref[...], axis=1, keepdims=True)
                + (jnp.sum(e3 * wvbr_ref[...], axis=1, keepdims=True)
                   / jnp.maximum(cnt_e, 1.0))
                + bv_ref[0, 0])
  out_ref[...] = ox + ov


def kernel(x, edge_index, edge_attr, batch, edge_batch, W1, b1, W2, b2,
           w_vz, b_vz):
  del edge_index
  batch = batch.astype(jnp.int32)
  edge_batch = edge_batch.astype(jnp.int32)
  zx = jnp.zeros((N_GRAPHS, D_FEAT), jnp.float32)
  ze = jnp.zeros((N_GRAPHS, 16), jnp.float32)

  sumx_p, sume3_p, cntx_p, cnte_p = _sc_segment_sums(
      x, batch, edge_attr.T, edge_batch, zx, ze)

  w1b_rep = jnp.repeat(W1[D_FEAT:, :], 16, axis=0)
  wvb_rep = jnp.repeat(w_vz[D_FEAT:], 16).reshape(1, -1)

  out = pl.pallas_call(
      _mlp_head,
      out_shape=jax.ShapeDtypeStruct((N_GRAPHS, 1), jnp.float32),
  )(sumx_p, sume3_p, cntx_p, cnte_p,
    W1[:D_FEAT, :], w1b_rep, b1.reshape(1, -1),
    W2.reshape(1, -1), b2.reshape(1, 1),
    w_vz[:D_FEAT].reshape(1, -1), wvb_rep,
    jnp.asarray(b_vz, jnp.float32).reshape(1, 1))
  return out
